# baseline (speedup 1.0000x reference)
import functools

import jax
import jax.numpy as jnp
from jax import lax
from jax.experimental import pallas as pl
from jax.experimental.pallas import tpu as pltpu
from jax.experimental.pallas import tpu_sc as plsc


def kernel(x, pe):
    T = x.shape[1]
    D = pe.shape[2]
    n = T * D

    info = plsc.get_sparse_core_info()
    nw = info.num_cores * info.num_subcores
    assert n % nw == 0 and (n // nw) % 8 == 0
    chunk = n // nw

    mesh = plsc.VectorSubcoreMesh(core_axis_name="c", subcore_axis_name="s")

    @functools.partial(
        pl.kernel,
        mesh=mesh,
        out_type=jax.ShapeDtypeStruct((n,), pe.dtype),
        scratch_types=[pltpu.VMEM((chunk,), pe.dtype)],
    )
    def sc_copy(pe_hbm, out_hbm, buf):
        wid = lax.axis_index("s") * info.num_cores + lax.axis_index("c")
        base = wid * chunk
        pltpu.sync_copy(pe_hbm.at[pl.ds(base, chunk)], buf)
        pltpu.sync_copy(buf, out_hbm.at[pl.ds(base, chunk)])

    out = sc_copy(pe.reshape(-1))
    return out.reshape(1, T, D)
